# baseline (device time: 33678 ns/iter reference)
import jax
import jax.numpy as jnp
from jax import lax
from jax.experimental import pallas as pl
from jax.experimental.pallas import tpu as pltpu

N_DEV = 8
HQ_PER = 4
DH = 64
BLK = 64

NBR_MASK = (1, 3, 4)
A_SLOT = (1, 3, 4)
B_SRC = (4, 1, 3)
B_SLOT = (5, 2, 7)
B_DEP = (2, 0, 1)
C_SRC = (7, 5, 2)
C_DEP = (2, 0, 1)
WQ_ROWS = ((0, 176), (176, 176), (352, 160))
WO_ROWS = ((0, 88), (88, 88), (176, 80))
COMPUTE_ORDER = (4, 1, 3, 7, 5, 2, 6)


def kernel(x, Wq, K_ext, V_ext, Wo):
    B_per, Sq, Dm = x.shape
    _, Hcols = Wq.shape
    _, Skv, Hq, _ = K_ext.shape

    my = lax.axis_index("i")
    K_my = lax.dynamic_slice_in_dim(K_ext, my * B_per, B_per, axis=0)
    V_my = lax.dynamic_slice_in_dim(V_ext, my * B_per, B_per, axis=0)
    K_r = jnp.transpose(K_my, (2, 0, 3, 1))
    V_r = jnp.transpose(V_my, (2, 0, 1, 3))
    Wq16 = Wq.astype(jnp.bfloat16)
    Wo16 = Wo.astype(jnp.bfloat16)

    def body(x_ref, wq_ref, k_ref, v_ref, wo_ref, out_ref,
             wq_all, wo_all, a_send, a_recv, b_send, b_recv, c_send, c_recv):
        my_pos = lax.axis_index("i")
        nbrs = [my_pos ^ m for m in NBR_MASK]

        bsem = pltpu.get_barrier_semaphore()
        for nb in nbrs:
            pl.semaphore_signal(
                bsem, inc=1, device_id=(nb,),
                device_id_type=pl.DeviceIdType.MESH,
            )
        pl.semaphore_wait(bsem, len(nbrs))

        def rdma(src, dst, ssem, rsem, peer):
            return pltpu.make_async_remote_copy(
                src_ref=src, dst_ref=dst, send_sem=ssem, recv_sem=rsem,
                device_id=(peer,), device_id_type=pl.DeviceIdType.MESH,
            )

        started = []

        for li in range(3):
            for t, (src, all_ref) in enumerate(((wq_ref, wq_all), (wo_ref, wo_all))):
                r = rdma(src, all_ref.at[A_SLOT[li]],
                         a_send.at[2 * li + t], a_recv.at[2 * li + t], nbrs[li])
                r.start()
                started.append(r)

        def wait_a(li, t):
            all_ref = (wq_all, wo_all)[t]
            rdma(wq_ref if t == 0 else wo_ref, all_ref.at[A_SLOT[li]],
                 a_send.at[0], a_recv.at[2 * li + t], my_pos).wait_recv()

        def wait_b(li, t):
            all_ref = (wq_all, wo_all)[t]
            rdma(wq_ref if t == 0 else wo_ref, all_ref.at[B_SLOT[li]],
                 b_send.at[0], b_recv.at[2 * li + t], my_pos).wait_recv()

        row_blk = lax.broadcasted_iota(jnp.int32, (Sq, Skv), 0) // BLK
        col_blk = lax.broadcasted_iota(jnp.int32, (Sq, Skv), 1) // BLK
        mask = col_blk <= row_blk

        def compute_chunk(m, wq_c, wo_c):
            g = my_pos ^ m
            for b in range(B_per):
                xb = x_ref[b].astype(jnp.bfloat16)
                q = jnp.dot(xb, wq_c, preferred_element_type=jnp.float32)
                ctx = []
                for hh in range(HQ_PER):
                    h = g * HQ_PER + hh
                    qh = q[:, hh * DH:(hh + 1) * DH].astype(jnp.bfloat16)
                    kt = k_ref[h, b].astype(jnp.bfloat16)
                    sc = jnp.dot(qh, kt, preferred_element_type=jnp.float32)
                    sc = jnp.where(mask, sc * 0.125, -1e9)
                    mx = jnp.max(sc, axis=1, keepdims=True)
                    w = jnp.exp(sc - mx)
                    w = (w / jnp.sum(w, axis=1, keepdims=True)).astype(jnp.bfloat16)
                    vv = v_ref[h, b].astype(jnp.bfloat16)
                    ctx.append(
                        jnp.dot(w, vv, preferred_element_type=jnp.float32)
                    )
                ctx = jnp.concatenate(ctx, axis=1).astype(jnp.bfloat16)
                contrib = jnp.dot(ctx, wo_c, preferred_element_type=jnp.float32)
                if m == 0:
                    out_ref[b] = contrib
                else:
                    out_ref[b] = out_ref[b] + contrib

        compute_chunk(0, wq_ref[...], wo_ref[...])

        for t, all_ref in enumerate((wq_all, wo_all)):
            for li in range(3):
                wait_a(B_DEP[li], t)
                r = rdma(all_ref.at[B_SRC[li]], all_ref.at[B_SLOT[li]],
                         b_send.at[2 * li + t], b_recv.at[2 * li + t], nbrs[li])
                r.start()
                started.append(r)

        for m in (4, 1, 3):
            compute_chunk(m, wq_all[m], wo_all[m])

        for t, (all_ref, rows) in enumerate(((wq_all, WQ_ROWS), (wo_all, WO_ROWS))):
            for li in range(3):
                wait_b(C_DEP[li], t)
                r0, rn = rows[li]
                r = rdma(all_ref.at[C_SRC[li], pl.ds(r0, rn)],
                         all_ref.at[6, pl.ds(r0, rn)],
                         c_send.at[2 * li + t], c_recv.at[2 * li + t], nbrs[li])
                r.start()
                started.append(r)

        for m in (7, 5, 2):
            compute_chunk(m, wq_all[m], wo_all[m])

        for li in range(3):
            qr0, qrn = WQ_ROWS[li]
            or0, orn = WO_ROWS[li]
            for t, (all_ref, (r0, rn)) in enumerate(
                ((wq_all, (qr0, qrn)), (wo_all, (or0, orn)))
            ):
                rdma(all_ref.at[C_SRC[li], pl.ds(r0, rn)],
                     all_ref.at[6, pl.ds(r0, rn)],
                     c_send.at[0], c_recv.at[2 * li + t], my_pos).wait_recv()
        compute_chunk(6, wq_all[6], wo_all[6])

        for r in started:
            r.wait_send()

    return pl.pallas_call(
        body,
        out_shape=jax.ShapeDtypeStruct((B_per, Sq, Dm), jnp.float32),
        in_specs=[pl.BlockSpec(memory_space=pltpu.VMEM)] * 5,
        out_specs=pl.BlockSpec(memory_space=pltpu.VMEM),
        scratch_shapes=[
            pltpu.VMEM((N_DEV, Dm, Hcols), jnp.bfloat16),
            pltpu.VMEM((N_DEV, Hcols, Dm), jnp.bfloat16),
            pltpu.SemaphoreType.DMA((6,)),
            pltpu.SemaphoreType.DMA((6,)),
            pltpu.SemaphoreType.DMA((6,)),
            pltpu.SemaphoreType.DMA((6,)),
            pltpu.SemaphoreType.DMA((6,)),
            pltpu.SemaphoreType.DMA((6,)),
        ],
        compiler_params=pltpu.CompilerParams(collective_id=0),
    )(x, Wq16, K_r, V_r, Wo16)


# device time: 31496 ns/iter; 1.0693x vs baseline; 1.0693x over previous
import jax
import jax.numpy as jnp
from jax import lax
from jax.experimental import pallas as pl
from jax.experimental.pallas import tpu as pltpu

N_DEV = 8
HQ_PER = 4
DH = 64
BLK = 64

NBR_MASK = (1, 3, 4)
A_SLOT = (1, 3, 4)
B_SRC = (4, 1, 3)
B_SLOT = (5, 2, 7)
B_DEP = (2, 0, 1)
C_SRC = (7, 5, 2)
C_DEP = (2, 0, 1)
WQ_ROWS = ((0, 176), (176, 176), (352, 160))
WO_ROWS = ((0, 88), (88, 88), (176, 80))
COMPUTE_ORDER = (4, 1, 3, 7, 5, 2, 6)


def kernel(x, Wq, K_ext, V_ext, Wo):
    B_per, Sq, Dm = x.shape
    _, Hcols = Wq.shape
    _, Skv, Hq, _ = K_ext.shape

    my = lax.axis_index("i")
    K_my = lax.dynamic_slice_in_dim(K_ext, my * B_per, B_per, axis=0)
    V_my = lax.dynamic_slice_in_dim(V_ext, my * B_per, B_per, axis=0)
    K_r = jnp.transpose(K_my, (2, 0, 3, 1))
    V_r = jnp.transpose(V_my, (2, 0, 1, 3))
    Wq16 = Wq.astype(jnp.bfloat16)
    Wo16 = Wo.astype(jnp.bfloat16)

    def body(x_ref, wq_ref, k_ref, v_ref, wo_ref, out_ref,
             wq_all, wo_all, a_send, a_recv, b_send, b_recv, c_send, c_recv):
        my_pos = lax.axis_index("i")
        nbrs = [my_pos ^ m for m in NBR_MASK]

        bsem = pltpu.get_barrier_semaphore()
        for nb in nbrs:
            pl.semaphore_signal(
                bsem, inc=1, device_id=(nb,),
                device_id_type=pl.DeviceIdType.MESH,
            )
        pl.semaphore_wait(bsem, len(nbrs))

        def rdma(src, dst, ssem, rsem, peer):
            return pltpu.make_async_remote_copy(
                src_ref=src, dst_ref=dst, send_sem=ssem, recv_sem=rsem,
                device_id=(peer,), device_id_type=pl.DeviceIdType.MESH,
            )

        started = []

        for li in range(3):
            for t, (src, all_ref) in enumerate(((wq_ref, wq_all), (wo_ref, wo_all))):
                r = rdma(src, all_ref.at[A_SLOT[li]],
                         a_send.at[2 * li + t], a_recv.at[2 * li + t], nbrs[li])
                r.start()
                started.append(r)

        def wait_a(li, t):
            all_ref = (wq_all, wo_all)[t]
            rdma(wq_ref if t == 0 else wo_ref, all_ref.at[A_SLOT[li]],
                 a_send.at[0], a_recv.at[2 * li + t], my_pos).wait_recv()

        def wait_b(li, t):
            all_ref = (wq_all, wo_all)[t]
            rdma(wq_ref if t == 0 else wo_ref, all_ref.at[B_SLOT[li]],
                 b_send.at[0], b_recv.at[2 * li + t], my_pos).wait_recv()

        row_blk = lax.broadcasted_iota(jnp.int32, (Sq, Skv), 0) // BLK
        col_blk = lax.broadcasted_iota(jnp.int32, (Sq, Skv), 1) // BLK
        mask = col_blk <= row_blk

        def compute_chunk(m, wq_c, wo_c):
            g = my_pos ^ m
            wq_c = wq_c.astype(jnp.float32)
            wo_c = wo_c.astype(jnp.float32)
            for b in range(B_per):
                q = jnp.dot(x_ref[b], wq_c, preferred_element_type=jnp.float32)
                ctx = []
                for hh in range(HQ_PER):
                    h = g * HQ_PER + hh
                    qh = q[:, hh * DH:(hh + 1) * DH]
                    kt = k_ref[h, b]
                    sc = jnp.dot(qh, kt, preferred_element_type=jnp.float32)
                    sc = jnp.where(mask, sc * 0.125, -1e9)
                    w = jnp.exp(sc)
                    w = w / jnp.sum(w, axis=1, keepdims=True)
                    ctx.append(
                        jnp.dot(w, v_ref[h, b], preferred_element_type=jnp.float32)
                    )
                ctx = jnp.concatenate(ctx, axis=1)
                contrib = jnp.dot(ctx, wo_c, preferred_element_type=jnp.float32)
                if m == 0:
                    out_ref[b] = contrib
                else:
                    out_ref[b] = out_ref[b] + contrib

        compute_chunk(0, wq_ref[...], wo_ref[...])

        for t, all_ref in enumerate((wq_all, wo_all)):
            for li in range(3):
                wait_a(B_DEP[li], t)
                r = rdma(all_ref.at[B_SRC[li]], all_ref.at[B_SLOT[li]],
                         b_send.at[2 * li + t], b_recv.at[2 * li + t], nbrs[li])
                r.start()
                started.append(r)

        def launch_c(t):
            all_ref, rows = ((wq_all, WQ_ROWS), (wo_all, WO_ROWS))[t]
            for li in range(3):
                wait_b(C_DEP[li], t)
                r0, rn = rows[li]
                r = rdma(all_ref.at[C_SRC[li], pl.ds(r0, rn)],
                         all_ref.at[6, pl.ds(r0, rn)],
                         c_send.at[2 * li + t], c_recv.at[2 * li + t], nbrs[li])
                r.start()
                started.append(r)

        compute_chunk(4, wq_all[4], wo_all[4])
        launch_c(0)
        compute_chunk(1, wq_all[1], wo_all[1])
        compute_chunk(3, wq_all[3], wo_all[3])
        launch_c(1)

        for m in (7, 5, 2):
            compute_chunk(m, wq_all[m], wo_all[m])

        for li in range(3):
            qr0, qrn = WQ_ROWS[li]
            or0, orn = WO_ROWS[li]
            for t, (all_ref, (r0, rn)) in enumerate(
                ((wq_all, (qr0, qrn)), (wo_all, (or0, orn)))
            ):
                rdma(all_ref.at[C_SRC[li], pl.ds(r0, rn)],
                     all_ref.at[6, pl.ds(r0, rn)],
                     c_send.at[0], c_recv.at[2 * li + t], my_pos).wait_recv()
        compute_chunk(6, wq_all[6], wo_all[6])

        for r in started:
            r.wait_send()

    return pl.pallas_call(
        body,
        out_shape=jax.ShapeDtypeStruct((B_per, Sq, Dm), jnp.float32),
        in_specs=[pl.BlockSpec(memory_space=pltpu.VMEM)] * 5,
        out_specs=pl.BlockSpec(memory_space=pltpu.VMEM),
        scratch_shapes=[
            pltpu.VMEM((N_DEV, Dm, Hcols), jnp.bfloat16),
            pltpu.VMEM((N_DEV, Hcols, Dm), jnp.bfloat16),
            pltpu.SemaphoreType.DMA((6,)),
            pltpu.SemaphoreType.DMA((6,)),
            pltpu.SemaphoreType.DMA((6,)),
            pltpu.SemaphoreType.DMA((6,)),
            pltpu.SemaphoreType.DMA((6,)),
            pltpu.SemaphoreType.DMA((6,)),
        ],
        compiler_params=pltpu.CompilerParams(collective_id=0),
    )(x, Wq16, K_r, V_r, Wo16)


# device time: 27934 ns/iter; 1.2056x vs baseline; 1.1275x over previous
import jax
import jax.numpy as jnp
from jax import lax
from jax.experimental import pallas as pl
from jax.experimental.pallas import tpu as pltpu

N_DEV = 8
HQ_PER = 4
DH = 64
BLK = 64

NBR_MASK = (1, 3, 4)
A_SLOT = (1, 3, 4)
B_SRC = (4, 1, 3)
B_SLOT = (5, 2, 7)
B_DEP = (2, 0, 1)
C_SRC = (7, 5, 2)
C_DEP = (2, 0, 1)
WQ_ROWS = ((0, 176), (176, 176), (352, 160))
WO_ROWS = ((0, 88), (88, 88), (176, 80))
COMPUTE_ORDER = (4, 1, 3, 7, 5, 2, 6)


def kernel(x, Wq, K_ext, V_ext, Wo):
    B_per, Sq, Dm = x.shape
    _, Hcols = Wq.shape
    _, Skv, Hq, _ = K_ext.shape

    my = lax.axis_index("i")
    K_my = lax.dynamic_slice_in_dim(K_ext, my * B_per, B_per, axis=0)
    V_my = lax.dynamic_slice_in_dim(V_ext, my * B_per, B_per, axis=0)
    K_r = jnp.transpose(K_my, (2, 0, 3, 1))
    V_r = jnp.transpose(V_my, (2, 0, 1, 3))
    Wq16 = Wq.astype(jnp.bfloat16)
    Wo16 = Wo.astype(jnp.bfloat16)

    def body(x_ref, wq_ref, k_ref, v_ref, wo_ref, out_ref,
             wq_all, wo_all, ctx_buf,
             a_send, a_recv, b_send, b_recv, c_send, c_recv):
        my_pos = lax.axis_index("i")
        nbrs = [my_pos ^ m for m in NBR_MASK]

        bsem = pltpu.get_barrier_semaphore()
        for nb in nbrs:
            pl.semaphore_signal(
                bsem, inc=1, device_id=(nb,),
                device_id_type=pl.DeviceIdType.MESH,
            )
        pl.semaphore_wait(bsem, len(nbrs))

        def rdma(src, dst, ssem, rsem, peer):
            return pltpu.make_async_remote_copy(
                src_ref=src, dst_ref=dst, send_sem=ssem, recv_sem=rsem,
                device_id=(peer,), device_id_type=pl.DeviceIdType.MESH,
            )

        started = []

        for li in range(3):
            for t, (src, all_ref) in enumerate(((wq_ref, wq_all), (wo_ref, wo_all))):
                r = rdma(src, all_ref.at[A_SLOT[li]],
                         a_send.at[2 * li + t], a_recv.at[2 * li + t], nbrs[li])
                r.start()
                started.append(r)

        def wait_a(li, t):
            all_ref = (wq_all, wo_all)[t]
            rdma(wq_ref if t == 0 else wo_ref, all_ref.at[A_SLOT[li]],
                 a_send.at[0], a_recv.at[2 * li + t], my_pos).wait_recv()

        def wait_b(li, t):
            all_ref = (wq_all, wo_all)[t]
            rdma(wq_ref if t == 0 else wo_ref, all_ref.at[B_SLOT[li]],
                 b_send.at[0], b_recv.at[2 * li + t], my_pos).wait_recv()

        row_blk = lax.broadcasted_iota(jnp.int32, (Sq, Skv), 0) // BLK
        col_blk = lax.broadcasted_iota(jnp.int32, (Sq, Skv), 1) // BLK
        mask = col_blk <= row_blk

        def compute_chunk(m, wq_c, wo_c):
            g = my_pos ^ m
            wq_c = wq_c.astype(jnp.float32)
            wo_c = wo_c.astype(jnp.float32)
            for b in range(B_per):
                q = jnp.dot(x_ref[b], wq_c, preferred_element_type=jnp.float32)
                ctx = []
                for hh in range(HQ_PER):
                    h = g * HQ_PER + hh
                    qh = q[:, hh * DH:(hh + 1) * DH]
                    kt = k_ref[h, b]
                    sc = jnp.dot(qh, kt, preferred_element_type=jnp.float32)
                    sc = jnp.where(mask, sc * 0.125, -1e9)
                    w = jnp.exp(sc)
                    w = w / jnp.sum(w, axis=1, keepdims=True)
                    ctx.append(
                        jnp.dot(w, v_ref[h, b], preferred_element_type=jnp.float32)
                    )
                ctx = jnp.concatenate(ctx, axis=1)
                contrib = jnp.dot(ctx, wo_c, preferred_element_type=jnp.float32)
                if m == 0:
                    out_ref[b] = contrib
                else:
                    out_ref[b] = out_ref[b] + contrib

        def stage1(m):
            g = my_pos ^ m
            wq_c = wq_all[m].astype(jnp.float32)
            for b in range(B_per):
                q = jnp.dot(x_ref[b], wq_c, preferred_element_type=jnp.float32)
                ctx = []
                for hh in range(HQ_PER):
                    h = g * HQ_PER + hh
                    qh = q[:, hh * DH:(hh + 1) * DH]
                    kt = k_ref[h, b]
                    sc = jnp.dot(qh, kt, preferred_element_type=jnp.float32)
                    sc = jnp.where(mask, sc * 0.125, -1e9)
                    w = jnp.exp(sc)
                    w = w / jnp.sum(w, axis=1, keepdims=True)
                    ctx.append(
                        jnp.dot(w, v_ref[h, b], preferred_element_type=jnp.float32)
                    )
                ctx_buf[m, b] = jnp.concatenate(ctx, axis=1)

        def stage2(m):
            wo_c = wo_all[m].astype(jnp.float32)
            for b in range(B_per):
                out_ref[b] = out_ref[b] + jnp.dot(
                    ctx_buf[m, b], wo_c, preferred_element_type=jnp.float32
                )

        compute_chunk(0, wq_ref[...], wo_ref[...])

        def launch_b(t):
            all_ref = (wq_all, wo_all)[t]
            for li in range(3):
                wait_a(B_DEP[li], t)
                r = rdma(all_ref.at[B_SRC[li]], all_ref.at[B_SLOT[li]],
                         b_send.at[2 * li + t], b_recv.at[2 * li + t], nbrs[li])
                r.start()
                started.append(r)

        def launch_c(t):
            all_ref, rows = ((wq_all, WQ_ROWS), (wo_all, WO_ROWS))[t]
            for li in range(3):
                wait_b(C_DEP[li], t)
                r0, rn = rows[li]
                r = rdma(all_ref.at[C_SRC[li], pl.ds(r0, rn)],
                         all_ref.at[6, pl.ds(r0, rn)],
                         c_send.at[2 * li + t], c_recv.at[2 * li + t], nbrs[li])
                r.start()
                started.append(r)

        def wait_c(t):
            all_ref, rows = ((wq_all, WQ_ROWS), (wo_all, WO_ROWS))[t]
            for li in range(3):
                r0, rn = rows[li]
                rdma(all_ref.at[C_SRC[li], pl.ds(r0, rn)],
                     all_ref.at[6, pl.ds(r0, rn)],
                     c_send.at[0], c_recv.at[2 * li + t], my_pos).wait_recv()

        launch_b(0)
        stage1(4)
        stage1(1)
        stage1(3)
        launch_b(1)
        stage2(4)
        stage2(1)
        stage2(3)
        launch_c(0)
        stage1(7)
        stage1(5)
        stage1(2)
        launch_c(1)
        stage2(7)
        stage2(5)
        stage2(2)
        wait_c(0)
        stage1(6)
        wait_c(1)
        stage2(6)

        for r in started:
            r.wait_send()

    return pl.pallas_call(
        body,
        out_shape=jax.ShapeDtypeStruct((B_per, Sq, Dm), jnp.float32),
        in_specs=[pl.BlockSpec(memory_space=pltpu.VMEM)] * 5,
        out_specs=pl.BlockSpec(memory_space=pltpu.VMEM),
        scratch_shapes=[
            pltpu.VMEM((N_DEV, Dm, Hcols), jnp.bfloat16),
            pltpu.VMEM((N_DEV, Hcols, Dm), jnp.bfloat16),
            pltpu.VMEM((N_DEV, B_per, Sq, Hcols), jnp.float32),
            pltpu.SemaphoreType.DMA((6,)),
            pltpu.SemaphoreType.DMA((6,)),
            pltpu.SemaphoreType.DMA((6,)),
            pltpu.SemaphoreType.DMA((6,)),
            pltpu.SemaphoreType.DMA((6,)),
            pltpu.SemaphoreType.DMA((6,)),
        ],
        compiler_params=pltpu.CompilerParams(collective_id=0),
    )(x, Wq16, K_r, V_r, Wo16)


# device time: 27889 ns/iter; 1.2076x vs baseline; 1.0016x over previous
import jax
import jax.numpy as jnp
from jax import lax
from jax.experimental import pallas as pl
from jax.experimental.pallas import tpu as pltpu

N_DEV = 8
HQ_PER = 4
DH = 64
BLK = 64

NBR_MASK = (1, 3, 4)
A_SLOT = (1, 3, 4)
B_SRC = (4, 1, 3)
B_SLOT = (5, 2, 7)
B_DEP = (2, 0, 1)
C_SRC = (7, 5, 2)
C_DEP = (2, 0, 1)
WQ_ROWS = ((0, 176), (176, 176), (352, 160))
WO_ROWS = ((0, 88), (88, 88), (176, 80))
COMPUTE_ORDER = (4, 1, 3, 7, 5, 2, 6)


def kernel(x, Wq, K_ext, V_ext, Wo):
    B_per, Sq, Dm = x.shape
    _, Hcols = Wq.shape
    _, Skv, Hq, _ = K_ext.shape

    my = lax.axis_index("i")
    K_my = lax.dynamic_slice_in_dim(K_ext, my * B_per, B_per, axis=0)
    V_my = lax.dynamic_slice_in_dim(V_ext, my * B_per, B_per, axis=0)
    K_r = jnp.transpose(K_my, (2, 0, 3, 1))
    V_r = jnp.transpose(V_my, (2, 0, 1, 3))
    Wq16 = (Wq * 0.125).astype(jnp.bfloat16)
    Wo16 = Wo.astype(jnp.bfloat16)

    def body(x_ref, wq_ref, k_ref, v_ref, wo_ref, out_ref,
             wq_all, wo_all, ctx_buf,
             a_send, a_recv, b_send, b_recv, c_send, c_recv):
        my_pos = lax.axis_index("i")
        nbrs = [my_pos ^ m for m in NBR_MASK]

        bsem = pltpu.get_barrier_semaphore()
        for nb in nbrs:
            pl.semaphore_signal(
                bsem, inc=1, device_id=(nb,),
                device_id_type=pl.DeviceIdType.MESH,
            )
        pl.semaphore_wait(bsem, len(nbrs))

        def rdma(src, dst, ssem, rsem, peer):
            return pltpu.make_async_remote_copy(
                src_ref=src, dst_ref=dst, send_sem=ssem, recv_sem=rsem,
                device_id=(peer,), device_id_type=pl.DeviceIdType.MESH,
            )

        started = []

        for li in range(3):
            for t, (src, all_ref) in enumerate(((wq_ref, wq_all), (wo_ref, wo_all))):
                r = rdma(src, all_ref.at[A_SLOT[li]],
                         a_send.at[2 * li + t], a_recv.at[2 * li + t], nbrs[li])
                r.start()
                started.append(r)

        def wait_a(li, t):
            all_ref = (wq_all, wo_all)[t]
            rdma(wq_ref if t == 0 else wo_ref, all_ref.at[A_SLOT[li]],
                 a_send.at[0], a_recv.at[2 * li + t], my_pos).wait_recv()

        def wait_b(li, t):
            all_ref = (wq_all, wo_all)[t]
            rdma(wq_ref if t == 0 else wo_ref, all_ref.at[B_SLOT[li]],
                 b_send.at[0], b_recv.at[2 * li + t], my_pos).wait_recv()

        row_blk = lax.broadcasted_iota(jnp.int32, (Sq, Skv), 0) // BLK
        col_blk = lax.broadcasted_iota(jnp.int32, (Sq, Skv), 1) // BLK
        mask = col_blk <= row_blk

        def compute_chunk(m, wq_c, wo_c):
            g = my_pos ^ m
            wq_c = wq_c.astype(jnp.float32)
            wo_c = wo_c.astype(jnp.float32)
            for b in range(B_per):
                q = jnp.dot(x_ref[b], wq_c, preferred_element_type=jnp.float32)
                ctx = []
                for hh in range(HQ_PER):
                    h = g * HQ_PER + hh
                    qh = q[:, hh * DH:(hh + 1) * DH]
                    kt = k_ref[h, b]
                    sc = jnp.dot(qh, kt, preferred_element_type=jnp.float32)
                    sc = jnp.where(mask, sc, -1e9)
                    w = jnp.exp(sc)
                    w = w / jnp.sum(w, axis=1, keepdims=True)
                    ctx.append(
                        jnp.dot(w, v_ref[h, b], preferred_element_type=jnp.float32)
                    )
                ctx = jnp.concatenate(ctx, axis=1)
                contrib = jnp.dot(ctx, wo_c, preferred_element_type=jnp.float32)
                if m == 0:
                    out_ref[b] = contrib
                else:
                    out_ref[b] = out_ref[b] + contrib

        def stage1(m):
            g = my_pos ^ m
            wq_c = wq_all[m].astype(jnp.float32)
            for b in range(B_per):
                q = jnp.dot(x_ref[b], wq_c, preferred_element_type=jnp.float32)
                ctx = []
                for hh in range(HQ_PER):
                    h = g * HQ_PER + hh
                    qh = q[:, hh * DH:(hh + 1) * DH]
                    kt = k_ref[h, b]
                    sc = jnp.dot(qh, kt, preferred_element_type=jnp.float32)
                    sc = jnp.where(mask, sc, -1e9)
                    w = jnp.exp(sc)
                    w = w / jnp.sum(w, axis=1, keepdims=True)
                    ctx.append(
                        jnp.dot(w, v_ref[h, b], preferred_element_type=jnp.float32)
                    )
                ctx_buf[m, b] = jnp.concatenate(ctx, axis=1)

        def stage2(m):
            wo_c = wo_all[m].astype(jnp.float32)
            for b in range(B_per):
                out_ref[b] = out_ref[b] + jnp.dot(
                    ctx_buf[m, b], wo_c, preferred_element_type=jnp.float32
                )

        compute_chunk(0, wq_ref[...], wo_ref[...])

        def launch_b(t):
            all_ref = (wq_all, wo_all)[t]
            for li in range(3):
                wait_a(B_DEP[li], t)
                r = rdma(all_ref.at[B_SRC[li]], all_ref.at[B_SLOT[li]],
                         b_send.at[2 * li + t], b_recv.at[2 * li + t], nbrs[li])
                r.start()
                started.append(r)

        def launch_c(t):
            all_ref, rows = ((wq_all, WQ_ROWS), (wo_all, WO_ROWS))[t]
            for li in range(3):
                wait_b(C_DEP[li], t)
                r0, rn = rows[li]
                r = rdma(all_ref.at[C_SRC[li], pl.ds(r0, rn)],
                         all_ref.at[6, pl.ds(r0, rn)],
                         c_send.at[2 * li + t], c_recv.at[2 * li + t], nbrs[li])
                r.start()
                started.append(r)

        def wait_c(t):
            all_ref, rows = ((wq_all, WQ_ROWS), (wo_all, WO_ROWS))[t]
            for li in range(3):
                r0, rn = rows[li]
                rdma(all_ref.at[C_SRC[li], pl.ds(r0, rn)],
                     all_ref.at[6, pl.ds(r0, rn)],
                     c_send.at[0], c_recv.at[2 * li + t], my_pos).wait_recv()

        launch_b(0)
        stage1(4)
        stage1(1)
        stage1(3)
        launch_b(1)
        stage2(4)
        stage2(1)
        stage2(3)
        launch_c(0)
        stage1(7)
        stage1(5)
        stage1(2)
        launch_c(1)
        stage2(7)
        stage2(5)
        stage2(2)
        wait_c(0)
        stage1(6)
        wait_c(1)
        stage2(6)

        for r in started:
            r.wait_send()

    return pl.pallas_call(
        body,
        out_shape=jax.ShapeDtypeStruct((B_per, Sq, Dm), jnp.float32),
        in_specs=[pl.BlockSpec(memory_space=pltpu.VMEM)] * 5,
        out_specs=pl.BlockSpec(memory_space=pltpu.VMEM),
        scratch_shapes=[
            pltpu.VMEM((N_DEV, Dm, Hcols), jnp.bfloat16),
            pltpu.VMEM((N_DEV, Hcols, Dm), jnp.bfloat16),
            pltpu.VMEM((N_DEV, B_per, Sq, Hcols), jnp.float32),
            pltpu.SemaphoreType.DMA((6,)),
            pltpu.SemaphoreType.DMA((6,)),
            pltpu.SemaphoreType.DMA((6,)),
            pltpu.SemaphoreType.DMA((6,)),
            pltpu.SemaphoreType.DMA((6,)),
            pltpu.SemaphoreType.DMA((6,)),
        ],
        compiler_params=pltpu.CompilerParams(collective_id=0),
    )(x, Wq16, K_r, V_r, Wo16)
